# baseline (speedup 1.0000x reference)
import functools

import jax
import jax.numpy as jnp
from jax import lax
from jax.experimental import pallas as pl
from jax.experimental.pallas import tpu as pltpu
from jax.experimental.pallas import tpu_sc as plsc

K_TOP = 100
K_LR = 0.1


def _ordered_key_u32(s):
    u = lax.bitcast_convert_type(s, jnp.uint32)
    flip = jnp.where(u >= jnp.uint32(0x80000000),
                     jnp.uint32(0xFFFFFFFF), jnp.uint32(0x80000000))
    return u ^ flip


def _mask_body(scores_ref, mw_ref):
    s = scores_ref[...]
    n, bb = s.shape
    key = _ordered_key_u32(s)

    p = jnp.zeros((bb,), dtype=jnp.uint32)
    for bit in range(31, -1, -1):
        cand = p | jnp.uint32(1 << bit)
        cnt = jnp.sum((key >= cand[None, :]).astype(jnp.int32), axis=0)
        p = jnp.where(cnt >= K_TOP, cand, p)

    count_gt = jnp.sum((key > p[None, :]).astype(jnp.int32), axis=0)
    need = K_TOP - count_gt

    eq = key == p[None, :]
    iota = lax.broadcasted_iota(jnp.int32, (n, bb), 0)
    lo = jnp.zeros((bb,), dtype=jnp.int32)
    hi = jnp.full((bb,), n, dtype=jnp.int32)
    for _ in range(10):
        mid = (lo + hi) // 2
        c = jnp.sum((eq & (iota < mid[None, :])).astype(jnp.int32), axis=0)
        cond = c >= need
        hi = jnp.where(cond, mid, hi)
        lo = jnp.where(cond, lo, mid)

    m = (key > p[None, :]) | (eq & (iota < hi[None, :]))
    w = jnp.left_shift(jnp.int32(1), iota % 32)
    wv = jnp.where(m, w, jnp.int32(0))
    words = jnp.sum(wv.reshape(n // 32, 32, bb), axis=1)
    mw_ref[...] = words


def _sc_body(mw_hbm, act_hbm, oldloc_hbm, oldscale_hbm, out_hbm,
             mw_v, idxbuf, rowidx, rows, oldloc_v, oldscale_v,
             locout, scaleout, sem):
    nc = 2
    wid = lax.axis_index("s") * nc + lax.axis_index("c")
    bpw = 64
    base = wid * bpw
    ebase = base * 16

    pltpu.sync_copy(mw_hbm.at[pl.ds(base * 32, bpw * 32)], mw_v)
    pltpu.sync_copy(oldloc_hbm.at[pl.ds(ebase, bpw * 16)], oldloc_v)
    pltpu.sync_copy(oldscale_hbm.at[pl.ds(ebase, bpw * 16)], oldscale_v)

    iota16 = lax.broadcasted_iota(jnp.int32, (16,), 0)
    zeros16 = jnp.zeros((16,), jnp.int32)

    def batch_body(j, carry):
        w0 = mw_v[pl.ds(j * 32, 16)]
        w1 = mw_v[pl.ds(j * 32 + 16, 16)]
        for r in range(8):
            idxbuf[pl.ds(r * 16, 16)] = zeros16
        off = jnp.int32(0)
        for bit in range(32):
            m0 = ((w0 >> bit) & 1) == 1
            vals0 = iota16 * 32 + bit
            pos0 = off + plsc.cumsum(m0.astype(jnp.int32)) - 1
            plsc.store_scatter(idxbuf, [pos0], vals0, mask=m0)
            off = off + plsc.all_reduce_population_count(m0)[0]
            m1 = ((w1 >> bit) & 1) == 1
            vals1 = iota16 * 32 + (512 + bit)
            pos1 = off + plsc.cumsum(m1.astype(jnp.int32)) - 1
            plsc.store_scatter(idxbuf, [pos1], vals1, mask=m1)
            off = off + plsc.all_reduce_population_count(m1)[0]
        bglob = base + j
        for r in range(7):
            rowidx[pl.ds(r * 16, 16)] = idxbuf[pl.ds(r * 16, 16)] * 2048 + bglob
        pltpu.async_copy(act_hbm.at[rowidx], rows, sem).wait()
        s = jnp.zeros((16,), jnp.float32)
        q = jnp.zeros((16,), jnp.float32)
        for r in range(K_TOP):
            v = rows[r]
            s = s + v
            q = q + v * v
        mean = s * (1.0 / K_TOP)
        var = (q - s * mean) * (1.0 / (K_TOP - 1))
        x = var + 1e-6
        i0 = lax.bitcast_convert_type(x, jnp.int32)
        y = lax.bitcast_convert_type((i0 >> 1) + jnp.int32(0x1FBD1DF5),
                                     jnp.float32)
        for _ in range(4):
            y = 0.5 * (y + x / y)
        sl = pl.ds(j * 16, 16)
        locout[sl] = (1.0 - K_LR) * oldloc_v[sl] + K_LR * mean
        scaleout[sl] = (1.0 - K_LR) * oldscale_v[sl] + K_LR * y
        return carry

    lax.fori_loop(0, bpw, batch_body, jnp.int32(0))

    pltpu.sync_copy(locout, out_hbm.at[0, pl.ds(ebase, bpw * 16)])
    pltpu.sync_copy(scaleout, out_hbm.at[1, pl.ds(ebase, bpw * 16)])


def _make_sc_call(B, V):
    mesh = plsc.VectorSubcoreMesh(core_axis_name="c", subcore_axis_name="s")
    bpw = B // 32
    return pl.kernel(
        _sc_body,
        mesh=mesh,
        compiler_params=pltpu.CompilerParams(
            needs_layout_passes=False, use_tc_tiling_on_sc=False),
        out_type=jax.ShapeDtypeStruct((2, B * V), jnp.float32),
        scratch_types=[
            pltpu.VMEM((bpw * 32,), jnp.int32),
            pltpu.VMEM((128,), jnp.int32),
            pltpu.VMEM((112,), jnp.int32),
            pltpu.VMEM((112, 16), jnp.float32),
            pltpu.VMEM((bpw * 16,), jnp.float32),
            pltpu.VMEM((bpw * 16,), jnp.float32),
            pltpu.VMEM((bpw * 16,), jnp.float32),
            pltpu.VMEM((bpw * 16,), jnp.float32),
            pltpu.SemaphoreType.DMA,
        ],
    )


@jax.jit
def kernel(obj, actions, old_loc, old_scale):
    N, B, V = actions.shape
    scores = obj[..., 0]
    BB = 512

    mw = pl.pallas_call(
        _mask_body,
        grid=(B // BB,),
        in_specs=[pl.BlockSpec((N, BB), lambda bi: (0, bi))],
        out_specs=pl.BlockSpec((N // 32, BB), lambda bi: (0, bi)),
        out_shape=jax.ShapeDtypeStruct((N // 32, B), jnp.int32),
    )(scores)

    out = _make_sc_call(B, V)(
        mw.T.reshape(B * (N // 32)),
        actions.reshape(N * B, V),
        old_loc.reshape(B * V),
        old_scale.reshape(B * V),
    )
    return out.reshape(2, B, V)


# speedup vs baseline: 6.6336x; 6.6336x over previous
import functools

import jax
import jax.numpy as jnp
from jax import lax
from jax.experimental import pallas as pl
from jax.experimental.pallas import tpu as pltpu

K_TOP = 100
K_LR = 0.1


def _ordered_key_u32(s):
    u = lax.bitcast_convert_type(s, jnp.uint32)
    flip = jnp.where(u >= jnp.uint32(0x80000000),
                     jnp.uint32(0xFFFFFFFF), jnp.uint32(0x80000000))
    return u ^ flip


def _thresh_body(scores_ref, thr_ref, idxthr_ref):
    s = scores_ref[...]
    n, bb = s.shape
    key = _ordered_key_u32(s)

    p = jnp.zeros((bb,), dtype=jnp.uint32)
    for bit in range(31, -1, -1):
        cand = p | jnp.uint32(1 << bit)
        cnt = jnp.sum((key >= cand[None, :]).astype(jnp.int32), axis=0)
        p = jnp.where(cnt >= K_TOP, cand, p)

    count_gt = jnp.sum((key > p[None, :]).astype(jnp.int32), axis=0)
    need = K_TOP - count_gt

    eq = key == p[None, :]
    iota = lax.broadcasted_iota(jnp.int32, (n, bb), 0)
    lo = jnp.zeros((bb,), dtype=jnp.int32)
    hi = jnp.full((bb,), n, dtype=jnp.int32)
    for _ in range(10):
        mid = (lo + hi) // 2
        c = jnp.sum((eq & (iota < mid[None, :])).astype(jnp.int32), axis=0)
        cond = c >= need
        hi = jnp.where(cond, mid, hi)
        lo = jnp.where(cond, lo, mid)

    thr_ref[...] = p
    idxthr_ref[...] = hi


def _accum_body(scores_ref, at_ref, thr_ref, idxthr_ref,
                oldloc_ref, oldscale_ref, out_ref, acc_ref, accsq_ref):
    ni = pl.program_id(1)
    nn = pl.num_programs(1)

    s = scores_ref[...]
    nb, bb = s.shape
    key = _ordered_key_u32(s)
    thr = thr_ref[...][None, :]
    idxthr = idxthr_ref[...][None, :]
    iota = ni * nb + lax.broadcasted_iota(jnp.int32, (nb, bb), 0)
    m = (key > thr) | ((key == thr) & (iota < idxthr))
    mf = m.astype(jnp.float32)

    a = at_ref[...]
    am = a * mf[:, None, :]
    psum = jnp.sum(am, axis=0)
    psumsq = jnp.sum(am * am, axis=0)

    @pl.when(ni == 0)
    def _init():
        acc_ref[...] = psum
        accsq_ref[...] = psumsq

    @pl.when(ni > 0)
    def _acc():
        acc_ref[...] += psum
        accsq_ref[...] += psumsq

    @pl.when(ni == nn - 1)
    def _finalize():
        tot = acc_ref[...]
        totsq = accsq_ref[...]
        mean = tot * (1.0 / K_TOP)
        var = (totsq - tot * mean) * (1.0 / (K_TOP - 1))
        scale = jnp.sqrt(var + 1e-6)
        new_loc = (1.0 - K_LR) * oldloc_ref[...] + K_LR * mean
        new_scale = (1.0 - K_LR) * oldscale_ref[...] + K_LR * scale
        out_ref[...] = jnp.stack([new_loc, new_scale], axis=0)


@jax.jit
def kernel(obj, actions, old_loc, old_scale):
    N, B, V = actions.shape
    scores = obj[..., 0]
    at = jnp.moveaxis(actions, -1, 1)
    oldloc_t = old_loc.T
    oldscale_t = old_scale.T
    BB = 512
    NB = 128

    thr, idxthr = pl.pallas_call(
        _thresh_body,
        grid=(B // BB,),
        in_specs=[pl.BlockSpec((N, BB), lambda bi: (0, bi))],
        out_specs=[pl.BlockSpec((BB,), lambda bi: (bi,)),
                   pl.BlockSpec((BB,), lambda bi: (bi,))],
        out_shape=[jax.ShapeDtypeStruct((B,), jnp.uint32),
                   jax.ShapeDtypeStruct((B,), jnp.int32)],
    )(scores)

    out_t = pl.pallas_call(
        _accum_body,
        grid=(B // BB, N // NB),
        in_specs=[
            pl.BlockSpec((NB, BB), lambda bi, ni: (ni, bi)),
            pl.BlockSpec((NB, V, BB), lambda bi, ni: (ni, 0, bi)),
            pl.BlockSpec((BB,), lambda bi, ni: (bi,)),
            pl.BlockSpec((BB,), lambda bi, ni: (bi,)),
            pl.BlockSpec((V, BB), lambda bi, ni: (0, bi)),
            pl.BlockSpec((V, BB), lambda bi, ni: (0, bi)),
        ],
        out_specs=pl.BlockSpec((2, V, BB), lambda bi, ni: (0, 0, bi)),
        out_shape=jax.ShapeDtypeStruct((2, V, B), jnp.float32),
        scratch_shapes=[pltpu.VMEM((V, BB), jnp.float32),
                        pltpu.VMEM((V, BB), jnp.float32)],
    )(scores, at, thr, idxthr, oldloc_t, oldscale_t)

    return jnp.moveaxis(out_t, 1, -1)


# speedup vs baseline: 6.8628x; 1.0345x over previous
import functools

import jax
import jax.numpy as jnp
from jax import lax
from jax.experimental import pallas as pl
from jax.experimental.pallas import tpu as pltpu

K_TOP = 100
K_LR = 0.1


def _ordered_key_u32(s):
    u = lax.bitcast_convert_type(s, jnp.uint32)
    flip = jnp.where(u >= jnp.uint32(0x80000000),
                     jnp.uint32(0xFFFFFFFF), jnp.uint32(0x80000000))
    return u ^ flip


def _fused_body(scores_ref, at_ref, oldloc_ref, oldscale_ref, out_ref,
                thr_ref, idxthr_ref, acc_ref, accsq_ref):
    ni = pl.program_id(1)
    nn = pl.num_programs(1)
    nb = at_ref.shape[0]

    @pl.when(ni == 0)
    def _thresholds():
        s = scores_ref[...]
        n, bb = s.shape
        key = _ordered_key_u32(s)

        p = jnp.zeros((bb,), dtype=jnp.uint32)
        for bit in range(31, -1, -1):
            cand = p | jnp.uint32(1 << bit)
            cnt = jnp.sum((key >= cand[None, :]).astype(jnp.int32), axis=0)
            p = jnp.where(cnt >= K_TOP, cand, p)

        count_gt = jnp.sum((key > p[None, :]).astype(jnp.int32), axis=0)
        need = K_TOP - count_gt

        eq = key == p[None, :]
        iota = lax.broadcasted_iota(jnp.int32, (n, bb), 0)
        lo = jnp.zeros((bb,), dtype=jnp.int32)
        hi = jnp.full((bb,), n, dtype=jnp.int32)
        for _ in range(10):
            mid = (lo + hi) // 2
            c = jnp.sum((eq & (iota < mid[None, :])).astype(jnp.int32), axis=0)
            cond = c >= need
            hi = jnp.where(cond, mid, hi)
            lo = jnp.where(cond, lo, mid)

        thr_ref[...] = p
        idxthr_ref[...] = hi
        acc_ref[...] = jnp.zeros_like(acc_ref)
        accsq_ref[...] = jnp.zeros_like(accsq_ref)

    s = scores_ref[pl.ds(ni * nb, nb), :]
    bb = s.shape[1]
    key = _ordered_key_u32(s)
    thr = thr_ref[...][None, :]
    idxthr = idxthr_ref[...][None, :]
    iota = ni * nb + lax.broadcasted_iota(jnp.int32, (nb, bb), 0)
    m = (key > thr) | ((key == thr) & (iota < idxthr))
    mf = m.astype(jnp.float32)

    a = at_ref[...]
    am = a * mf[:, None, :]
    acc_ref[...] += jnp.sum(am, axis=0)
    accsq_ref[...] += jnp.sum(am * am, axis=0)

    @pl.when(ni == nn - 1)
    def _finalize():
        tot = acc_ref[...]
        totsq = accsq_ref[...]
        mean = tot * (1.0 / K_TOP)
        var = (totsq - tot * mean) * (1.0 / (K_TOP - 1))
        scale = jnp.sqrt(var + 1e-6)
        new_loc = (1.0 - K_LR) * oldloc_ref[...] + K_LR * mean
        new_scale = (1.0 - K_LR) * oldscale_ref[...] + K_LR * scale
        out_ref[...] = jnp.stack([new_loc, new_scale], axis=0)


@jax.jit
def kernel(obj, actions, old_loc, old_scale):
    N, B, V = actions.shape
    scores = obj[..., 0]
    at = jnp.moveaxis(actions, -1, 1)
    oldloc_t = old_loc.T
    oldscale_t = old_scale.T
    BB = 512
    NB = 128

    out_t = pl.pallas_call(
        _fused_body,
        grid=(B // BB, N // NB),
        in_specs=[
            pl.BlockSpec((N, BB), lambda bi, ni: (0, bi)),
            pl.BlockSpec((NB, V, BB), lambda bi, ni: (ni, 0, bi)),
            pl.BlockSpec((V, BB), lambda bi, ni: (0, bi)),
            pl.BlockSpec((V, BB), lambda bi, ni: (0, bi)),
        ],
        out_specs=pl.BlockSpec((2, V, BB), lambda bi, ni: (0, 0, bi)),
        out_shape=jax.ShapeDtypeStruct((2, V, B), jnp.float32),
        scratch_shapes=[pltpu.VMEM((BB,), jnp.uint32),
                        pltpu.VMEM((BB,), jnp.int32),
                        pltpu.VMEM((V, BB), jnp.float32),
                        pltpu.VMEM((V, BB), jnp.float32)],
    )(scores, at, oldloc_t, oldscale_t)

    return jnp.moveaxis(out_t, 1, -1)
